# Initial kernel scaffold; baseline (speedup 1.0000x reference)
import jax
import jax.numpy as jnp
from jax.experimental import pallas as pl


def kernel(landmarks, flow):
    raise NotImplementedError("write your pallas kernel here")


# speedup vs baseline: 5.7749x; 5.7749x over previous
import dataclasses
import functools

import jax
import jax.numpy as jnp
from jax import lax
from jax.experimental import pallas as pl
from jax.experimental.pallas import tpu as pltpu
from jax.experimental.pallas import tpu_sc as plsc

_B, _N, _W, _H = 4, 256, 512, 512
_WH = _W * _H
_L = 16
_NW = 32
_LPW = (_B * _N) // _NW
_CH = _LPW // _L
_ROWS = _B * 2 * _WH // _L


def _floor_f32(x):
    t = x.astype(jnp.int32).astype(jnp.float32)
    return jnp.where(t > x, t - 1.0, t)


def _sc_compiler_params():
    cp = pltpu.CompilerParams()
    if "needs_layout_passes" in pltpu.CompilerParams.__dataclass_fields__:
        cp = dataclasses.replace(cp, needs_layout_passes=False)
    if "use_tc_tiling_on_sc" in pltpu.CompilerParams.__dataclass_fields__:
        cp = dataclasses.replace(cp, use_tc_tiling_on_sc=False)
    return cp


def _sc_partials(lm_t, flow_tbl):
    mesh = plsc.VectorSubcoreMesh(core_axis_name="c", subcore_axis_name="s")

    @functools.partial(
        pl.kernel,
        compiler_params=_sc_compiler_params(),
        out_type=jax.ShapeDtypeStruct((_NW, _L), jnp.float32),
        mesh=mesh,
        scratch_types=[
            pltpu.VMEM((4 * _LPW,), jnp.float32),
            pltpu.VMEM((8 * _L,), jnp.int32),
            pltpu.VMEM((8 * _L,), jnp.int32),
            pltpu.VMEM((8 * _L, _L), jnp.float32),
            pltpu.VMEM((8 * _L, _L), jnp.float32),
            pltpu.VMEM((_L,), jnp.float32),
            pltpu.SemaphoreType.DMA,
            pltpu.SemaphoreType.DMA,
        ],
    )
    def kern(lm_hbm, flow_hbm, out_hbm, lm_v, rows0, rows1, vals0, vals1,
             part_v, sem0, sem1):
        wid = lax.axis_index("c") * 16 + lax.axis_index("s")
        bofs = lax.div(wid, 8) * (2 * _WH)
        pltpu.sync_copy(lm_hbm.at[wid], lm_v)
        iota = lax.iota(jnp.int32, _L)

        rows_refs = (rows0, rows1)
        vals_refs = (vals0, vals1)
        sems = (sem0, sem1)
        lanes = [[None] * 8 for _ in range(_CH)]
        held = [None] * _CH
        copies = [None] * _CH
        for c in range(_CH):
            x1 = lm_v[pl.ds(0 * _LPW + c * _L, _L)]
            y1 = lm_v[pl.ds(1 * _LPW + c * _L, _L)]
            x1_d = _floor_f32(x1)
            y1_d = _floor_f32(y1)
            x1_u = x1_d + 1.0
            y1_u = y1_d + 1.0
            xd = jnp.minimum(jnp.maximum(x1_d.astype(jnp.int32), 0), _W - 1)
            yd = jnp.minimum(jnp.maximum(y1_d.astype(jnp.int32), 0), _H - 1)
            xu = jnp.minimum(jnp.maximum(x1_u.astype(jnp.int32), 0), _W - 1)
            yu = jnp.minimum(jnp.maximum(y1_u.astype(jnp.int32), 0), _H - 1)
            mask = (x1_u < float(_W)) & (y1_u < float(_H))
            wa = (x1 - x1_d) * (y1 - y1_d)
            wb = (x1_u - x1) * (y1_u - x1)
            wc = (x1_u - x1) * (y1 - y1_d)
            wd = (x1 - x1_d) * (y1_u - x1)
            held[c] = (x1, y1, mask, wa, wb, wc, wd)
            xs = (xd, xu, xu, xd)
            ys = (yd, yu, yd, yu)
            for ch in range(2):
                for j in range(4):
                    k = ch * 4 + j
                    f = bofs + ch * _WH + xs[j] * _H + ys[j]
                    rows_refs[c][pl.ds(k * _L, _L)] = jnp.right_shift(f, 4)
                    lanes[c][k] = jnp.bitwise_and(f, _L - 1)
            copies[c] = pltpu.async_copy(
                flow_hbm.at[rows_refs[c]], vals_refs[c], sems[c])

        acc = jnp.zeros((_L,), jnp.float32)
        for c in range(_CH):
            copies[c].wait()
            x1, y1, mask, wa, wb, wc, wd = held[c]
            x2 = lm_v[pl.ds(2 * _LPW + c * _L, _L)]
            y2 = lm_v[pl.ds(3 * _LPW + c * _L, _L)]
            v = [None] * 8
            for k in range(8):
                pos = k * _L + iota
                v[k] = plsc.load_gather(vals_refs[c], [pos, lanes[c][k]])
            o_x = v[0] * wa + v[1] * wb + v[2] * wc + v[3] * wd
            o_y = v[4] * wa + v[5] * wb + v[6] * wc + v[7] * wd
            dx = x1 + o_x - x2
            dy = y1 + o_y - y2
            per = dx * dx + dy * dy
            acc = acc + jnp.where(mask, per, 0.0)

        part_v[...] = acc
        pltpu.sync_copy(part_v, out_hbm.at[wid])

    return kern(lm_t, flow_tbl)


def _reduce_tc(parts):
    def body(p_ref, o_ref):
        o_ref[...] = jnp.sum(p_ref[...]).reshape(1, 1) * (1.0 / (2.0 * _B))

    return pl.pallas_call(
        body,
        out_shape=jax.ShapeDtypeStruct((1, 1), jnp.float32),
    )(parts)


def kernel(landmarks, flow):
    lm_t = (landmarks.reshape(_NW, _LPW, 4)
            .transpose(0, 2, 1)
            .reshape(_NW, 4 * _LPW))
    flow_tbl = flow.reshape(_ROWS, _L)
    parts = _sc_partials(lm_t, flow_tbl)
    return _reduce_tc(parts)[0, 0]
